# Initial kernel scaffold; baseline (speedup 1.0000x reference)
import jax
import jax.numpy as jnp
from jax.experimental import pallas as pl


def kernel(x, edge_index, Wl0, bl0, Wr0, g0, be0, Wl1, bl1, Wr1, g1, be1, Wl2, bl2, Wr2, g2, be2):
    raise NotImplementedError("write your pallas kernel here")


# speedup vs baseline: 2.2336x; 2.2336x over previous
import functools

import jax
import jax.numpy as jnp
from jax import lax
from jax.experimental import pallas as pl
from jax.experimental.pallas import tpu as pltpu
from jax.experimental.pallas import tpu_sc as plsc

N = 10000
E = 320000
D_IN = 128
D_HID = 128
D_OUT = 64

NC = 2
NS = 16
NW = NC * NS
K = 128
NCH = 80
EPW = NCH * K
E_PAD = NW * EPW
NPAD = N + 8
RPT = 624
REM = N - RPT * NS

_f32 = jnp.float32


def _make_sc_agg(D):
  mesh = plsc.VectorSubcoreMesh(core_axis_name="c", subcore_axis_name="s")
  out_type = [jax.ShapeDtypeStruct((NC, N, D), _f32)]
  scratch = [
      pltpu.VMEM((NCH, K), jnp.int32),
      pltpu.VMEM((NCH, K), jnp.int32),
      pltpu.VMEM((K, D), _f32),
      pltpu.VMEM_SHARED((NPAD, D), _f32),
      pltpu.SemaphoreType.DMA,
  ]

  def body(z, src3, dst3, zrow, agg_out, srcv, dstv, rowsv, acc, sem):
    c = lax.axis_index("c")
    s = lax.axis_index("s")
    wid = c * NS + s
    pltpu.sync_copy(src3.at[wid], srcv)
    pltpu.sync_copy(dst3.at[wid], dstv)
    off = pl.multiple_of(s * RPT, 8)
    pltpu.sync_copy(zrow.at[pl.ds(off, RPT)], acc.at[pl.ds(off, RPT)])

    @pl.when(s == NS - 1)
    def _():
      pltpu.sync_copy(zrow.at[pl.ds(0, REM + 8)],
                      acc.at[pl.ds(RPT * NS, REM + 8)])

    plsc.subcore_barrier()

    def chunk(i, carry):
      pltpu.async_copy(z.at[srcv.at[i]], rowsv, sem).wait()
      pltpu.sync_copy(rowsv, acc.at[dstv.at[i]], add=True)
      return carry

    lax.fori_loop(0, NCH, chunk, 0)
    plsc.subcore_barrier()
    pltpu.sync_copy(acc.at[pl.ds(off, RPT)], agg_out.at[c, pl.ds(off, RPT)])

    @pl.when(s == NS - 1)
    def _():
      pltpu.sync_copy(acc.at[pl.ds(RPT * NS, REM)],
                      agg_out.at[c, pl.ds(RPT * NS, REM)])

  return pl.kernel(body, mesh=mesh, out_type=out_type, scratch_types=scratch)


_sc_agg_128 = functools.lru_cache(maxsize=None)(
    lambda: _make_sc_agg(D_HID))


BN = 1000


def _row_spec(d):
  return pl.BlockSpec((BN, d), lambda i: (i, 0))


def _full_spec(shape):
  return pl.BlockSpec(shape, lambda i: (0, 0))


def _dot(a, b):
  return jnp.dot(a, b, preferred_element_type=_f32,
                 precision=lax.Precision.HIGHEST)


def _tc_pre(x, wl, wr, bl):
  d_in, d_out = wl.shape

  def body(x_r, wl_r, wr_r, bl_r, z_r, r_r):
    xv = x_r[...]
    z_r[...] = _dot(xv, wl_r[...])
    r_r[...] = _dot(xv, wr_r[...]) + bl_r[...]

  return pl.pallas_call(
      body,
      grid=(N // BN,),
      in_specs=[_row_spec(d_in), _full_spec(wl.shape), _full_spec(wr.shape),
                _full_spec((1, d_out))],
      out_specs=[_row_spec(d_out), _row_spec(d_out)],
      out_shape=[jax.ShapeDtypeStruct((N, d_out), _f32)] * 2,
  )(x, wl, wr, bl.reshape(1, -1))


def _ln(pre, g, be):
  mu = jnp.mean(pre, axis=-1, keepdims=True)
  dv = pre - mu
  var = jnp.mean(dv * dv, axis=-1, keepdims=True)
  return dv * lax.rsqrt(var + 1e-5) * g + be


def _tc_mid(agg, cnta, cntb, r, g, be, wl, wr, bl, emit_h):
  d = r.shape[1]
  dz = d if emit_h else wl.shape[1]
  dr = wr.shape[1]

  def body(aa_r, ab_r, ca_r, cb_r, r_r, g_r, be_r, wl_r, wr_r, bl_r,
           z_r, rn_r):
    cnt = jnp.maximum(ca_r[...] + cb_r[...], 1.0)
    pre = (aa_r[...] + ab_r[...]) / cnt + r_r[...]
    h = jnp.maximum(_ln(pre, g_r[...], be_r[...]), 0.0)
    if emit_h:
      z_r[...] = h
    else:
      z_r[...] = _dot(h, wl_r[...])
    rn_r[...] = _dot(h, wr_r[...]) + bl_r[...]

  return pl.pallas_call(
      body,
      grid=(N // BN,),
      in_specs=[_row_spec(d), _row_spec(d),
                pl.BlockSpec((BN, 1), lambda i: (i, 0)),
                pl.BlockSpec((BN, 1), lambda i: (i, 0)),
                _row_spec(d), _full_spec((1, d)), _full_spec((1, d)),
                _full_spec(wl.shape), _full_spec(wr.shape),
                _full_spec((1, dr))],
      out_specs=[_row_spec(dz), _row_spec(dr)],
      out_shape=[jax.ShapeDtypeStruct((N, dz), _f32),
                 jax.ShapeDtypeStruct((N, dr), _f32)],
  )(agg[0], agg[1], cnta, cntb, r, g.reshape(1, -1), be.reshape(1, -1),
    wl, wr, bl.reshape(1, -1))


def _tc_post(agg, cnta, cntb, r, wl, g, be):
  d = agg.shape[2]
  do = r.shape[1]

  def body(aa_r, ab_r, ca_r, cb_r, r_r, wl_r, g_r, be_r, o_r):
    cnt = jnp.maximum(ca_r[...] + cb_r[...], 1.0)
    mean = (aa_r[...] + ab_r[...]) / cnt
    pre = _dot(mean, wl_r[...]) + r_r[...]
    h = _ln(pre, g_r[...], be_r[...])
    nrm = jnp.sqrt(jnp.sum(h * h, axis=-1, keepdims=True))
    o_r[...] = h / jnp.maximum(nrm, 1e-12)

  return pl.pallas_call(
      body,
      grid=(N // BN,),
      in_specs=[_row_spec(d), _row_spec(d),
                pl.BlockSpec((BN, 1), lambda i: (i, 0)),
                pl.BlockSpec((BN, 1), lambda i: (i, 0)),
                _row_spec(do), _full_spec(wl.shape),
                _full_spec((1, do)), _full_spec((1, do))],
      out_specs=_row_spec(do),
      out_shape=jax.ShapeDtypeStruct((N, do), _f32),
  )(agg[0], agg[1], cnta, cntb, r, wl, g.reshape(1, -1), be.reshape(1, -1))


def kernel(x, edge_index, Wl0, bl0, Wr0, g0, be0, Wl1, bl1, Wr1, g1, be1,
           Wl2, bl2, Wr2, g2, be2):
  npad = E_PAD - E
  src3 = jnp.concatenate(
      [edge_index[0], jnp.zeros((npad,), jnp.int32)]).reshape(NW, NCH, K)
  dst3 = jnp.concatenate(
      [edge_index[1], jnp.full((npad,), N, jnp.int32)]).reshape(NW, NCH, K)
  zrow = jnp.zeros((N, D_HID), _f32)
  ones_tab = jnp.ones((N, D_HID), _f32)
  sc_agg = _sc_agg_128()

  (cntw,) = sc_agg(ones_tab, src3, dst3, zrow)
  cnta = cntw[0, :, 0:1]
  cntb = cntw[1, :, 0:1]

  z0, r0 = _tc_pre(x, Wl0, Wr0, bl0)
  (agg0,) = sc_agg(z0, src3, dst3, zrow)
  z1, r1 = _tc_mid(agg0, cnta, cntb, r0, g0, be0, Wl1, Wr1, bl1,
                   emit_h=False)

  (agg1,) = sc_agg(z1, src3, dst3, zrow)
  h2, r2 = _tc_mid(agg1, cnta, cntb, r1, g1, be1, Wl2, Wr2, bl2,
                   emit_h=True)

  (agg2,) = sc_agg(h2, src3, dst3, zrow)
  return _tc_post(agg2, cnta, cntb, r2, Wl2, g2, be2)


# speedup vs baseline: 3.0828x; 1.3802x over previous
import functools

import jax
import jax.numpy as jnp
from jax import lax
from jax.experimental import pallas as pl
from jax.experimental.pallas import tpu as pltpu
from jax.experimental.pallas import tpu_sc as plsc

N = 10000
E = 320000
D_IN = 128
D_HID = 128
D_OUT = 64

NC = 2
NS = 16
NW = NC * NS
K = 128
NCH = 80
EPW = NCH * K
E_PAD = NW * EPW
NPAD = N + 8
RPT = 624
REM = N - RPT * NS

_f32 = jnp.float32


NPH = 2
CPP = NCH // NPH


def _zero_acc(s, zrow, acc):
  off = pl.multiple_of(s * RPT, 8)
  pltpu.sync_copy(zrow.at[pl.ds(off, RPT)], acc.at[pl.ds(off, RPT)])

  @pl.when(s == NS - 1)
  def _():
    pltpu.sync_copy(zrow.at[pl.ds(0, REM + 8)],
                    acc.at[pl.ds(RPT * NS, REM + 8)])

  return off


def _copy_out(c, s, off, acc, out):
  pltpu.sync_copy(acc.at[pl.ds(off, RPT)], out.at[c, pl.ds(off, RPT)])

  @pl.when(s == NS - 1)
  def _():
    pltpu.sync_copy(acc.at[pl.ds(RPT * NS, REM)],
                    out.at[c, pl.ds(RPT * NS, REM)])


def _make_sc_agg(D):
  mesh = plsc.VectorSubcoreMesh(core_axis_name="c", subcore_axis_name="s")
  out_type = [jax.ShapeDtypeStruct((NC, N, D), _f32)]
  scratch = [
      pltpu.VMEM((CPP, K), jnp.int32),
      pltpu.VMEM((CPP, K), jnp.int32),
      pltpu.VMEM((2, K, D), _f32),
      pltpu.VMEM_SHARED((NPAD, D), _f32),
      pltpu.SemaphoreType.DMA,
      pltpu.SemaphoreType.DMA,
  ]

  def body(z, src3, dst3, zrow, agg_out, srcv, dstv, rowsv, acc, s0, s1):
    c = lax.axis_index("c")
    s = lax.axis_index("s")
    wid = c * NS + s
    sems = (s0, s1)
    off = _zero_acc(s, zrow, acc)
    plsc.subcore_barrier()

    for p in range(NPH):
      pltpu.sync_copy(src3.at[wid, pl.ds(p * CPP, CPP)], srcv)
      pltpu.sync_copy(dst3.at[wid, pl.ds(p * CPP, CPP)], dstv)
      pltpu.async_copy(z.at[srcv.at[0]], rowsv.at[0], sems[0])
      pltpu.async_copy(z.at[srcv.at[1]], rowsv.at[1], sems[1])

      def pair(j, carry):
        for b in range(2):
          i = j * 2 + b
          pltpu.make_async_copy(z.at[srcv.at[i]], rowsv.at[b],
                                sems[b]).wait()
          pltpu.sync_copy(rowsv.at[b], acc.at[dstv.at[i]], add=True)

          @pl.when(i + 2 < CPP)
          def _():
            pltpu.async_copy(z.at[srcv.at[i + 2]], rowsv.at[b], sems[b])

        return carry

      lax.fori_loop(0, CPP // 2, pair, 0)

    plsc.subcore_barrier()
    _copy_out(c, s, off, acc, agg_out)

  return pl.kernel(body, mesh=mesh, out_type=out_type, scratch_types=scratch)


def _make_sc_cnt():
  mesh = plsc.VectorSubcoreMesh(core_axis_name="c", subcore_axis_name="s")
  out_type = [jax.ShapeDtypeStruct((NC, N, D_HID), _f32)]
  scratch = [
      pltpu.VMEM((NCH, K), jnp.int32),
      pltpu.VMEM((K, D_HID), _f32),
      pltpu.VMEM_SHARED((NPAD, D_HID), _f32),
  ]

  def body(dst3, zrow, one_tab, cnt_out, dstv, onesv, acc):
    c = lax.axis_index("c")
    s = lax.axis_index("s")
    wid = c * NS + s
    pltpu.sync_copy(dst3.at[wid], dstv)
    pltpu.sync_copy(one_tab, onesv)
    off = _zero_acc(s, zrow, acc)
    plsc.subcore_barrier()

    def chunk(i, carry):
      pltpu.sync_copy(onesv, acc.at[dstv.at[i]], add=True)
      return carry

    lax.fori_loop(0, NCH, chunk, 0)
    plsc.subcore_barrier()
    _copy_out(c, s, off, acc, cnt_out)

  return pl.kernel(body, mesh=mesh, out_type=out_type, scratch_types=scratch)


_sc_agg_128 = functools.lru_cache(maxsize=None)(
    lambda: _make_sc_agg(D_HID))
_sc_cnt = functools.lru_cache(maxsize=None)(_make_sc_cnt)


BN = 1000


def _row_spec(d):
  return pl.BlockSpec((BN, d), lambda i: (i, 0))


def _full_spec(shape):
  return pl.BlockSpec(shape, lambda i: (0, 0))


def _dot(a, b):
  return jnp.dot(a, b, preferred_element_type=_f32,
                 precision=lax.Precision.HIGHEST)


def _tc_pre(x, wl, wr, bl):
  d_in, d_out = wl.shape

  def body(x_r, wl_r, wr_r, bl_r, z_r, r_r):
    xv = x_r[...]
    z_r[...] = _dot(xv, wl_r[...])
    r_r[...] = _dot(xv, wr_r[...]) + bl_r[...]

  return pl.pallas_call(
      body,
      grid=(N // BN,),
      in_specs=[_row_spec(d_in), _full_spec(wl.shape), _full_spec(wr.shape),
                _full_spec((1, d_out))],
      out_specs=[_row_spec(d_out), _row_spec(d_out)],
      out_shape=[jax.ShapeDtypeStruct((N, d_out), _f32)] * 2,
  )(x, wl, wr, bl.reshape(1, -1))


def _ln(pre, g, be):
  mu = jnp.mean(pre, axis=-1, keepdims=True)
  dv = pre - mu
  var = jnp.mean(dv * dv, axis=-1, keepdims=True)
  return dv / jnp.sqrt(var + 1e-5) * g + be


def _tc_mid(agg, cnta, cntb, r, g, be, wl, wr, bl, emit_h):
  d = r.shape[1]
  dz = d if emit_h else wl.shape[1]
  dr = wr.shape[1]

  def body(aa_r, ab_r, ca_r, cb_r, r_r, g_r, be_r, wl_r, wr_r, bl_r,
           z_r, rn_r):
    cnt = jnp.maximum(ca_r[...] + cb_r[...], 1.0)
    pre = (aa_r[...] + ab_r[...]) / cnt + r_r[...]
    h = jnp.maximum(_ln(pre, g_r[...], be_r[...]), 0.0)
    if emit_h:
      z_r[...] = h
    else:
      z_r[...] = _dot(h, wl_r[...])
    rn_r[...] = _dot(h, wr_r[...]) + bl_r[...]

  return pl.pallas_call(
      body,
      grid=(N // BN,),
      in_specs=[_row_spec(d), _row_spec(d),
                pl.BlockSpec((BN, 1), lambda i: (i, 0)),
                pl.BlockSpec((BN, 1), lambda i: (i, 0)),
                _row_spec(d), _full_spec((1, d)), _full_spec((1, d)),
                _full_spec(wl.shape), _full_spec(wr.shape),
                _full_spec((1, dr))],
      out_specs=[_row_spec(dz), _row_spec(dr)],
      out_shape=[jax.ShapeDtypeStruct((N, dz), _f32),
                 jax.ShapeDtypeStruct((N, dr), _f32)],
  )(agg[0], agg[1], cnta, cntb, r, g.reshape(1, -1), be.reshape(1, -1),
    wl, wr, bl.reshape(1, -1))


def _tc_post(agg, cnta, cntb, r, wl, g, be):
  d = agg.shape[2]
  do = r.shape[1]

  def body(aa_r, ab_r, ca_r, cb_r, r_r, wl_r, g_r, be_r, o_r):
    cnt = jnp.maximum(ca_r[...] + cb_r[...], 1.0)
    mean = (aa_r[...] + ab_r[...]) / cnt
    pre = _dot(mean, wl_r[...]) + r_r[...]
    h = _ln(pre, g_r[...], be_r[...])
    nrm = jnp.sqrt(jnp.sum(h * h, axis=-1, keepdims=True))
    o_r[...] = h / jnp.maximum(nrm, 1e-12)

  return pl.pallas_call(
      body,
      grid=(N // BN,),
      in_specs=[_row_spec(d), _row_spec(d),
                pl.BlockSpec((BN, 1), lambda i: (i, 0)),
                pl.BlockSpec((BN, 1), lambda i: (i, 0)),
                _row_spec(do), _full_spec(wl.shape),
                _full_spec((1, do)), _full_spec((1, do))],
      out_specs=_row_spec(do),
      out_shape=jax.ShapeDtypeStruct((N, do), _f32),
  )(agg[0], agg[1], cnta, cntb, r, wl, g.reshape(1, -1), be.reshape(1, -1))


def kernel(x, edge_index, Wl0, bl0, Wr0, g0, be0, Wl1, bl1, Wr1, g1, be1,
           Wl2, bl2, Wr2, g2, be2):
  npad = E_PAD - E
  src3 = jnp.concatenate(
      [edge_index[0], jnp.zeros((npad,), jnp.int32)]).reshape(NW, NCH, K)
  dst3 = jnp.concatenate(
      [edge_index[1], jnp.full((npad,), N, jnp.int32)]).reshape(NW, NCH, K)
  zrow = jnp.zeros((N, D_HID), _f32)
  ones_blk = jnp.ones((K, D_HID), _f32)
  sc_agg = _sc_agg_128()

  (cntw,) = _sc_cnt()(dst3, zrow, ones_blk)
  cnta = cntw[0, :, 0:1]
  cntb = cntw[1, :, 0:1]

  z0, r0 = _tc_pre(x, Wl0, Wr0, bl0)
  (agg0,) = sc_agg(z0, src3, dst3, zrow)
  z1, r1 = _tc_mid(agg0, cnta, cntb, r0, g0, be0, Wl1, Wr1, bl1,
                   emit_h=False)

  (agg1,) = sc_agg(z1, src3, dst3, zrow)
  h2, r2 = _tc_mid(agg1, cnta, cntb, r1, g1, be1, Wl2, Wr2, bl2,
                   emit_h=True)

  (agg2,) = sc_agg(h2, src3, dst3, zrow)
  return _tc_post(agg2, cnta, cntb, r2, Wl2, g2, be2)
